# Initial kernel scaffold; baseline (speedup 1.0000x reference)
import jax
import jax.numpy as jnp
from jax.experimental import pallas as pl


def kernel(X, edge_index, Wm0, bm0, Wm1, bm1, Wg0, bg0, Wg1, bg1):
    raise NotImplementedError("write your pallas kernel here")


# speedup vs baseline: 12.8720x; 12.8720x over previous
import functools

import jax
import jax.numpy as jnp
from jax import lax
from jax.experimental import pallas as pl
from jax.experimental.pallas import tpu as pltpu
from jax.experimental.pallas import tpu_sc as plsc

N = 10000
E = 320000
D = 128
NC = 2
NS = 16
NW = NC * NS
EW = E // NW
C = 80
NCH = EW // C
NPAD = 10240
RPS = NPAD // NS
ZROWS = 128
DPS = NPAD // NS

def _deg_body(dst_hbm, out_hbm, dst_v, ones_v, zb_v, acc_sh):
    cid = lax.axis_index("c")
    sid = lax.axis_index("s")
    wid = cid * NS + sid

    z16 = jnp.zeros((16,), jnp.float32)
    o16 = jnp.ones((16,), jnp.float32)

    def zrow(i, carry):
        zb_v[pl.ds(i * 16, 16)] = z16
        return carry

    lax.fori_loop(0, DPS // 16, zrow, 0)
    for j in range(C // 16):
        ones_v[pl.ds(j * 16, 16)] = o16
    pltpu.sync_copy(zb_v, acc_sh.at[pl.ds(sid * DPS, DPS)])
    plsc.subcore_barrier()

    def body(i, carry):
        base = wid * EW + i * C
        pltpu.sync_copy(dst_hbm.at[pl.ds(base, C)], dst_v)
        pltpu.sync_copy(ones_v, acc_sh.at[dst_v], add=True)
        return carry

    lax.fori_loop(0, NCH, body, 0)
    plsc.subcore_barrier()
    pltpu.sync_copy(
        acc_sh.at[pl.ds(sid * DPS, DPS)],
        out_hbm.at[pl.ds(cid * NPAD + sid * DPS, DPS)],
    )


def _edge_body(y_hbm, src_hbm, dst_hbm, out_hbm, src_v, dst_v, rows_v, zb_v, acc_sh, sem):
    cid = lax.axis_index("c")
    sid = lax.axis_index("s")
    wid = cid * NS + sid

    z16 = jnp.zeros((16,), jnp.float32)

    def zrow(i, carry):
        for j in range(D // 16):
            zb_v[i, pl.ds(j * 16, 16)] = z16
        return carry

    lax.fori_loop(0, ZROWS, zrow, 0)

    def zacc(k, carry):
        pltpu.sync_copy(zb_v, acc_sh.at[pl.ds(sid * RPS + k * ZROWS, ZROWS)])
        return carry

    lax.fori_loop(0, RPS // ZROWS, zacc, 0)
    plsc.subcore_barrier()

    def body(i, carry):
        base = wid * EW + i * C
        pltpu.sync_copy(src_hbm.at[pl.ds(base, C)], src_v)
        pltpu.sync_copy(dst_hbm.at[pl.ds(base, C)], dst_v)
        pltpu.async_copy(y_hbm.at[src_v], rows_v, sem).wait()
        pltpu.sync_copy(rows_v, acc_sh.at[dst_v], add=True)
        return carry

    lax.fori_loop(0, NCH, body, 0)
    plsc.subcore_barrier()
    pltpu.sync_copy(
        acc_sh.at[pl.ds(sid * RPS, RPS)],
        out_hbm.at[pl.ds(cid * NPAD + sid * RPS, RPS)],
    )


_sc_built = {}


def _deg_pass(dst):
    if "deg" not in _sc_built:
        mesh = plsc.VectorSubcoreMesh(
            core_axis_name="c", subcore_axis_name="s",
            num_cores=NC, num_subcores=NS)
        _sc_built["deg"] = pl.kernel(
            _deg_body,
            out_type=jax.ShapeDtypeStruct((NC * NPAD,), jnp.float32),
            mesh=mesh,
            scratch_types=[
                pltpu.VMEM((C,), jnp.int32),
                pltpu.VMEM((C,), jnp.float32),
                pltpu.VMEM((DPS,), jnp.float32),
                pltpu.VMEM_SHARED((NPAD,), jnp.float32),
            ],
        )
    return _sc_built["deg"](dst)


def _edge_pass(y, src, dst):
    if "edge" not in _sc_built:
        mesh = plsc.VectorSubcoreMesh(
            core_axis_name="c", subcore_axis_name="s",
            num_cores=NC, num_subcores=NS)
        _sc_built["edge"] = pl.kernel(
            _edge_body,
            out_type=jax.ShapeDtypeStruct((NC * NPAD, D), jnp.float32),
            mesh=mesh,
            scratch_types=[
                pltpu.VMEM((C,), jnp.int32),
                pltpu.VMEM((C,), jnp.int32),
                pltpu.VMEM((C, D), jnp.float32),
                pltpu.VMEM((ZROWS, D), jnp.float32),
                pltpu.VMEM_SHARED((NPAD, D), jnp.float32),
                pltpu.SemaphoreType.DMA,
            ],
        )
    return _sc_built["edge"](y, src, dst)


BR = 1000


def _tc_a_body(x_ref, d0_ref, d1_ref, wm0_ref, bm0_ref, wm1_ref, bm1_ref,
               wg0_ref, y_ref, dinv_ref):
    f32 = jnp.float32
    h = jnp.maximum(
        jnp.dot(x_ref[...], wm0_ref[...], preferred_element_type=f32) + bm0_ref[...], 0.0)
    m = jnp.dot(h, wm1_ref[...], preferred_element_type=f32) + bm1_ref[...]
    deg = d0_ref[...] + d1_ref[...] + 1.0
    dinv = lax.rsqrt(jnp.maximum(deg, 1e-12))
    y_ref[...] = dinv * jnp.dot(m, wg0_ref[...], preferred_element_type=f32)
    dinv_ref[...] = dinv


def _tc_b_body(z0_ref, z1_ref, y1_ref, dinv_ref, bg0_ref, wg1_ref, y2_ref):
    z = z0_ref[...] + z1_ref[...] + y1_ref[...]
    h1 = jnp.maximum(dinv_ref[...] * z + bg0_ref[...], 0.0)
    y2_ref[...] = dinv_ref[...] * jnp.dot(
        h1, wg1_ref[...], preferred_element_type=jnp.float32)


def _tc_c_body(z0_ref, z1_ref, y2_ref, dinv_ref, bg1_ref, out_ref):
    out_ref[...] = dinv_ref[...] * (
        z0_ref[...] + z1_ref[...] + y2_ref[...]) + bg1_ref[...]


def _row_spec(cols):
    return pl.BlockSpec((BR, cols), lambda i: (i, 0))


def _full_spec(r, c):
    return pl.BlockSpec((r, c), lambda i: (0, 0))


_tc_a = pl.pallas_call(
    _tc_a_body,
    grid=(N // BR,),
    in_specs=[
        _row_spec(D), _row_spec(1), _row_spec(1),
        _full_spec(128, 256), _full_spec(1, 256),
        _full_spec(256, 128), _full_spec(1, 128),
        _full_spec(128, 128),
    ],
    out_specs=[_row_spec(D), _row_spec(1)],
    out_shape=[
        jax.ShapeDtypeStruct((N, D), jnp.float32),
        jax.ShapeDtypeStruct((N, 1), jnp.float32),
    ],
)

_tc_b = pl.pallas_call(
    _tc_b_body,
    grid=(N // BR,),
    in_specs=[
        _row_spec(D), _row_spec(D), _row_spec(D), _row_spec(1),
        _full_spec(1, 128), _full_spec(128, 128),
    ],
    out_specs=_row_spec(D),
    out_shape=jax.ShapeDtypeStruct((N, D), jnp.float32),
)

_tc_c = pl.pallas_call(
    _tc_c_body,
    grid=(N // BR,),
    in_specs=[
        _row_spec(D), _row_spec(D), _row_spec(D), _row_spec(1),
        _full_spec(1, 128),
    ],
    out_specs=_row_spec(D),
    out_shape=jax.ShapeDtypeStruct((N, D), jnp.float32),
)


def kernel(X, edge_index, Wm0, bm0, Wm1, bm1, Wg0, bg0, Wg1, bg1):
    src = edge_index[0]
    dst = edge_index[1]

    degp = _deg_pass(dst)
    d0 = degp[0:N].reshape(N, 1)
    d1 = degp[NPAD:NPAD + N].reshape(N, 1)

    y1, dinv = _tc_a(X, d0, d1, Wm0, bm0.reshape(1, -1), Wm1,
                     bm1.reshape(1, -1), Wg0)

    zp1 = _edge_pass(y1, src, dst)
    y2 = _tc_b(zp1[:N], zp1[NPAD:NPAD + N], y1, dinv, bg0.reshape(1, -1), Wg1)

    zp2 = _edge_pass(y2, src, dst)
    out = _tc_c(zp2[:N], zp2[NPAD:NPAD + N], y2, dinv, bg1.reshape(1, -1))
    return out


# speedup vs baseline: 23.9841x; 1.8633x over previous
import functools

import jax
import jax.numpy as jnp
from jax import lax
from jax.experimental import pallas as pl
from jax.experimental.pallas import tpu as pltpu
from jax.experimental.pallas import tpu_sc as plsc

N = 10000
E = 320000
D = 128
NC = 2
NS = 16
NW = NC * NS
EW = E // NW
C = 80
NCH = EW // C
NPAD = 10240
RPS = NPAD // NS
ZROWS = 128
DPS = NPAD // NS

def _deg_body(dst_hbm, out_hbm, dst_v, ones_v, zb_v, acc_sh):
    cid = lax.axis_index("c")
    sid = lax.axis_index("s")
    wid = cid * NS + sid

    z16 = jnp.zeros((16,), jnp.float32)
    o16 = jnp.ones((16,), jnp.float32)

    def zrow(i, carry):
        zb_v[pl.ds(i * 16, 16)] = z16
        return carry

    lax.fori_loop(0, DPS // 16, zrow, 0)
    for j in range(C // 16):
        ones_v[pl.ds(j * 16, 16)] = o16
    pltpu.sync_copy(zb_v, acc_sh.at[pl.ds(sid * DPS, DPS)])
    plsc.subcore_barrier()

    def body(i, carry):
        base = wid * EW + i * C
        pltpu.sync_copy(dst_hbm.at[pl.ds(base, C)], dst_v)
        pltpu.sync_copy(ones_v, acc_sh.at[dst_v], add=True)
        return carry

    lax.fori_loop(0, NCH, body, 0)
    plsc.subcore_barrier()
    pltpu.sync_copy(
        acc_sh.at[pl.ds(sid * DPS, DPS)],
        out_hbm.at[pl.ds(cid * NPAD + sid * DPS, DPS)],
    )


def _edge_body(y_hbm, src_hbm, dst_hbm, out_hbm, src_all, dst_v0, dst_v1,
               rows_v0, rows_v1, zb_v, acc_sh, sem0, sem1):
    cid = lax.axis_index("c")
    sid = lax.axis_index("s")
    wid = cid * NS + sid

    z16 = jnp.zeros((16,), jnp.float32)

    def zrow(i, carry):
        for j in range(D // 16):
            zb_v[i, pl.ds(j * 16, 16)] = z16
        return carry

    lax.fori_loop(0, ZROWS, zrow, 0)

    def zacc(k, carry):
        pltpu.sync_copy(zb_v, acc_sh.at[pl.ds(sid * RPS + k * ZROWS, ZROWS)])
        return carry

    lax.fori_loop(0, RPS // ZROWS, zacc, 0)

    pltpu.sync_copy(src_hbm.at[pl.ds(wid * EW, EW)], src_all)
    plsc.subcore_barrier()

    def fire(i, buf, sem):
        pltpu.async_copy(y_hbm.at[src_all.at[pl.ds(i * C, C)]], buf, sem)

    fire(0, rows_v0, sem0)

    def body2(k, carry):
        i0 = 2 * k
        fire(i0 + 1, rows_v1, sem1)
        pltpu.sync_copy(dst_hbm.at[pl.ds(wid * EW + i0 * C, C)], dst_v0)
        pltpu.make_async_copy(y_hbm.at[src_all.at[pl.ds(i0 * C, C)]],
                              rows_v0, sem0).wait()
        pltpu.sync_copy(rows_v0, acc_sh.at[dst_v0], add=True)
        fire(i0 + 2, rows_v0, sem0)
        pltpu.sync_copy(dst_hbm.at[pl.ds(wid * EW + (i0 + 1) * C, C)], dst_v1)
        pltpu.make_async_copy(y_hbm.at[src_all.at[pl.ds((i0 + 1) * C, C)]],
                              rows_v1, sem1).wait()
        pltpu.sync_copy(rows_v1, acc_sh.at[dst_v1], add=True)
        return carry

    lax.fori_loop(0, (NCH - 1) // 2, body2, 0)
    pltpu.sync_copy(dst_hbm.at[pl.ds(wid * EW + (NCH - 1) * C, C)], dst_v0)
    pltpu.make_async_copy(y_hbm.at[src_all.at[pl.ds((NCH - 1) * C, C)]],
                          rows_v0, sem0).wait()
    pltpu.sync_copy(rows_v0, acc_sh.at[dst_v0], add=True)

    plsc.subcore_barrier()
    pltpu.sync_copy(
        acc_sh.at[pl.ds(sid * RPS, RPS)],
        out_hbm.at[pl.ds(cid * NPAD + sid * RPS, RPS)],
    )


_sc_built = {}


def _deg_pass(dst):
    if "deg" not in _sc_built:
        mesh = plsc.VectorSubcoreMesh(
            core_axis_name="c", subcore_axis_name="s",
            num_cores=NC, num_subcores=NS)
        _sc_built["deg"] = pl.kernel(
            _deg_body,
            out_type=jax.ShapeDtypeStruct((NC * NPAD,), jnp.float32),
            mesh=mesh,
            scratch_types=[
                pltpu.VMEM((C,), jnp.int32),
                pltpu.VMEM((C,), jnp.float32),
                pltpu.VMEM((DPS,), jnp.float32),
                pltpu.VMEM_SHARED((NPAD,), jnp.float32),
            ],
        )
    return _sc_built["deg"](dst)


def _edge_pass(y, src, dst):
    if "edge" not in _sc_built:
        mesh = plsc.VectorSubcoreMesh(
            core_axis_name="c", subcore_axis_name="s",
            num_cores=NC, num_subcores=NS)
        _sc_built["edge"] = pl.kernel(
            _edge_body,
            out_type=jax.ShapeDtypeStruct((NC * NPAD, D), jnp.float32),
            mesh=mesh,
            scratch_types=[
                pltpu.VMEM((EW,), jnp.int32),
                pltpu.VMEM((C,), jnp.int32),
                pltpu.VMEM((C,), jnp.int32),
                pltpu.VMEM((C, D), jnp.float32),
                pltpu.VMEM((C, D), jnp.float32),
                pltpu.VMEM((ZROWS, D), jnp.float32),
                pltpu.VMEM_SHARED((NPAD, D), jnp.float32),
                pltpu.SemaphoreType.DMA,
                pltpu.SemaphoreType.DMA,
            ],
        )
    return _sc_built["edge"](y, src, dst)


BR = 1000


def _tc_a_body(x_ref, d0_ref, d1_ref, wm0_ref, bm0_ref, wm1_ref, bm1_ref,
               wg0_ref, y_ref, dinv_ref):
    f32 = jnp.float32
    h = jnp.maximum(
        jnp.dot(x_ref[...], wm0_ref[...], preferred_element_type=f32) + bm0_ref[...], 0.0)
    m = jnp.dot(h, wm1_ref[...], preferred_element_type=f32) + bm1_ref[...]
    deg = d0_ref[...] + d1_ref[...] + 1.0
    dinv = lax.rsqrt(jnp.maximum(deg, 1e-12))
    y_ref[...] = dinv * jnp.dot(m, wg0_ref[...], preferred_element_type=f32)
    dinv_ref[...] = dinv


def _tc_b_body(z0_ref, z1_ref, y1_ref, dinv_ref, bg0_ref, wg1_ref, y2_ref):
    z = z0_ref[...] + z1_ref[...] + y1_ref[...]
    h1 = jnp.maximum(dinv_ref[...] * z + bg0_ref[...], 0.0)
    y2_ref[...] = dinv_ref[...] * jnp.dot(
        h1, wg1_ref[...], preferred_element_type=jnp.float32)


def _tc_c_body(z0_ref, z1_ref, y2_ref, dinv_ref, bg1_ref, out_ref):
    out_ref[...] = dinv_ref[...] * (
        z0_ref[...] + z1_ref[...] + y2_ref[...]) + bg1_ref[...]


def _row_spec(cols):
    return pl.BlockSpec((BR, cols), lambda i: (i, 0))


def _full_spec(r, c):
    return pl.BlockSpec((r, c), lambda i: (0, 0))


_tc_a = pl.pallas_call(
    _tc_a_body,
    grid=(N // BR,),
    in_specs=[
        _row_spec(D), _row_spec(1), _row_spec(1),
        _full_spec(128, 256), _full_spec(1, 256),
        _full_spec(256, 128), _full_spec(1, 128),
        _full_spec(128, 128),
    ],
    out_specs=[_row_spec(D), _row_spec(1)],
    out_shape=[
        jax.ShapeDtypeStruct((N, D), jnp.float32),
        jax.ShapeDtypeStruct((N, 1), jnp.float32),
    ],
)

_tc_b = pl.pallas_call(
    _tc_b_body,
    grid=(N // BR,),
    in_specs=[
        _row_spec(D), _row_spec(D), _row_spec(D), _row_spec(1),
        _full_spec(1, 128), _full_spec(128, 128),
    ],
    out_specs=_row_spec(D),
    out_shape=jax.ShapeDtypeStruct((N, D), jnp.float32),
)

_tc_c = pl.pallas_call(
    _tc_c_body,
    grid=(N // BR,),
    in_specs=[
        _row_spec(D), _row_spec(D), _row_spec(D), _row_spec(1),
        _full_spec(1, 128),
    ],
    out_specs=_row_spec(D),
    out_shape=jax.ShapeDtypeStruct((N, D), jnp.float32),
)


def kernel(X, edge_index, Wm0, bm0, Wm1, bm1, Wg0, bg0, Wg1, bg1):
    src = edge_index[0]
    dst = edge_index[1]

    degp = _deg_pass(dst)
    d0 = degp[0:N].reshape(N, 1)
    d1 = degp[NPAD:NPAD + N].reshape(N, 1)

    y1, dinv = _tc_a(X, d0, d1, Wm0, bm0.reshape(1, -1), Wm1,
                     bm1.reshape(1, -1), Wg0)

    zp1 = _edge_pass(y1, src, dst)
    y2 = _tc_b(zp1[:N], zp1[NPAD:NPAD + N], y1, dinv, bg0.reshape(1, -1), Wg1)

    zp2 = _edge_pass(y2, src, dst)
    out = _tc_c(zp2[:N], zp2[NPAD:NPAD + N], y2, dinv, bg1.reshape(1, -1))
    return out


# speedup vs baseline: 29.7301x; 1.2396x over previous
import functools

import jax
import jax.numpy as jnp
from jax import lax
from jax.experimental import pallas as pl
from jax.experimental.pallas import tpu as pltpu
from jax.experimental.pallas import tpu_sc as plsc

N = 10000
E = 320000
D = 128
NC = 2
NS = 16
NW = NC * NS
EW = E // NW
C = 80
NCH = EW // C
NPAD = 10240
RPS = NPAD // NS
ZROWS = 128
DPS = NPAD // NS

DRING = 5


def _deg_body(dst3_hbm, out_hbm, dst3_v, ones_v, zb_v, acc_sh, *sems):
    cid = lax.axis_index("c")
    sid = lax.axis_index("s")
    wid = cid * NS + sid

    z16 = jnp.zeros((16,), jnp.float32)
    o16 = jnp.ones((16,), jnp.float32)

    def zrow(i, carry):
        zb_v[pl.ds(i * 16, 16)] = z16
        return carry

    lax.fori_loop(0, DPS // 16, zrow, 0)
    for j in range(C // 16):
        ones_v[pl.ds(j * 16, 16)] = o16
    pltpu.sync_copy(zb_v, acc_sh.at[pl.ds(sid * DPS, DPS)])
    pltpu.sync_copy(dst3_hbm.at[wid], dst3_v)
    plsc.subcore_barrier()

    def fire(i, j):
        pltpu.async_copy(ones_v, acc_sh.at[dst3_v.at[i]], sems[j], add=True)

    def wait(j):
        pltpu.make_async_copy(ones_v, acc_sh.at[dst3_v.at[0]], sems[j]).wait()

    for j in range(DRING):
        fire(j, j)

    def body(k, carry):
        for j in range(DRING):
            wait(j)
            fire(k * DRING + j, j)
        return carry

    lax.fori_loop(1, NCH // DRING, body, 0)
    for j in range(DRING):
        wait(j)
    plsc.subcore_barrier()
    pltpu.sync_copy(
        acc_sh.at[pl.ds(sid * DPS, DPS)],
        out_hbm.at[pl.ds(cid * NPAD + sid * DPS, DPS)],
    )


def _edge_body(y_hbm, src_hbm, dst3_hbm, out_hbm, src_all, dst3_v,
               rows_v0, rows_v1, acc_sh, sem0, sem1):
    cid = lax.axis_index("c")
    sid = lax.axis_index("s")
    wid = cid * NS + sid

    z16 = jnp.zeros((16,), jnp.float32)

    def zrow(i, carry):
        for j in range(D // 16):
            rows_v0[i, pl.ds(j * 16, 16)] = z16
        return carry

    lax.fori_loop(0, C, zrow, 0)

    def zacc(k, carry):
        pltpu.sync_copy(rows_v0, acc_sh.at[pl.ds(sid * RPS + k * C, C)])
        return carry

    lax.fori_loop(0, RPS // C, zacc, 0)

    pltpu.sync_copy(src_hbm.at[pl.ds(wid * EW, EW)], src_all)
    pltpu.sync_copy(dst3_hbm.at[wid], dst3_v)
    plsc.subcore_barrier()

    def fire(i, buf, sem):
        pltpu.async_copy(y_hbm.at[src_all.at[pl.ds(i * C, C)]], buf, sem)

    def gwait(i, buf, sem):
        pltpu.make_async_copy(y_hbm.at[src_all.at[pl.ds(i * C, C)]],
                              buf, sem).wait()

    fire(0, rows_v0, sem0)

    def body2(k, carry):
        i0 = 2 * k
        fire(i0 + 1, rows_v1, sem1)
        gwait(i0, rows_v0, sem0)
        pltpu.sync_copy(rows_v0, acc_sh.at[dst3_v.at[i0]], add=True)
        fire(i0 + 2, rows_v0, sem0)
        gwait(i0 + 1, rows_v1, sem1)
        pltpu.sync_copy(rows_v1, acc_sh.at[dst3_v.at[i0 + 1]], add=True)
        return carry

    lax.fori_loop(0, (NCH - 1) // 2, body2, 0)
    gwait(NCH - 1, rows_v0, sem0)
    pltpu.sync_copy(rows_v0, acc_sh.at[dst3_v.at[NCH - 1]], add=True)

    plsc.subcore_barrier()
    pltpu.sync_copy(
        acc_sh.at[pl.ds(sid * RPS, RPS)],
        out_hbm.at[pl.ds(cid * NPAD + sid * RPS, RPS)],
    )


_sc_built = {}


def _deg_pass(dst):
    if "deg" not in _sc_built:
        mesh = plsc.VectorSubcoreMesh(
            core_axis_name="c", subcore_axis_name="s",
            num_cores=NC, num_subcores=NS)
        _sc_built["deg"] = pl.kernel(
            _deg_body,
            out_type=jax.ShapeDtypeStruct((NC * NPAD,), jnp.float32),
            mesh=mesh,
            scratch_types=[
                pltpu.VMEM((NCH, C), jnp.int32),
                pltpu.VMEM((C,), jnp.float32),
                pltpu.VMEM((DPS,), jnp.float32),
                pltpu.VMEM_SHARED((NPAD,), jnp.float32),
            ] + [pltpu.SemaphoreType.DMA] * DRING,
        )
    return _sc_built["deg"](dst.reshape(NW, NCH, C))


def _edge_pass(y, src, dst):
    if "edge" not in _sc_built:
        mesh = plsc.VectorSubcoreMesh(
            core_axis_name="c", subcore_axis_name="s",
            num_cores=NC, num_subcores=NS)
        _sc_built["edge"] = pl.kernel(
            _edge_body,
            out_type=jax.ShapeDtypeStruct((NC * NPAD, D), jnp.float32),
            mesh=mesh,
            scratch_types=[
                pltpu.VMEM((EW,), jnp.int32),
                pltpu.VMEM((NCH, C), jnp.int32),
                pltpu.VMEM((C, D), jnp.float32),
                pltpu.VMEM((C, D), jnp.float32),
                pltpu.VMEM_SHARED((NPAD, D), jnp.float32),
                pltpu.SemaphoreType.DMA,
                pltpu.SemaphoreType.DMA,
            ],
        )
    return _sc_built["edge"](y, src, dst.reshape(NW, NCH, C))


BR = 1000


def _tc_a_body(x_ref, d0_ref, d1_ref, wm0_ref, bm0_ref, wm1_ref, bm1_ref,
               wg0_ref, y_ref, dinv_ref):
    f32 = jnp.float32
    h = jnp.maximum(
        jnp.dot(x_ref[...], wm0_ref[...], preferred_element_type=f32) + bm0_ref[...], 0.0)
    m = jnp.dot(h, wm1_ref[...], preferred_element_type=f32) + bm1_ref[...]
    deg = d0_ref[...] + d1_ref[...] + 1.0
    dinv = lax.rsqrt(jnp.maximum(deg, 1e-12))
    y_ref[...] = dinv * jnp.dot(m, wg0_ref[...], preferred_element_type=f32)
    dinv_ref[...] = dinv


def _tc_b_body(z0_ref, z1_ref, y1_ref, dinv_ref, bg0_ref, wg1_ref, y2_ref):
    z = z0_ref[...] + z1_ref[...] + y1_ref[...]
    h1 = jnp.maximum(dinv_ref[...] * z + bg0_ref[...], 0.0)
    y2_ref[...] = dinv_ref[...] * jnp.dot(
        h1, wg1_ref[...], preferred_element_type=jnp.float32)


def _tc_c_body(z0_ref, z1_ref, y2_ref, dinv_ref, bg1_ref, out_ref):
    out_ref[...] = dinv_ref[...] * (
        z0_ref[...] + z1_ref[...] + y2_ref[...]) + bg1_ref[...]


def _row_spec(cols):
    return pl.BlockSpec((BR, cols), lambda i: (i, 0))


def _full_spec(r, c):
    return pl.BlockSpec((r, c), lambda i: (0, 0))


_tc_a = pl.pallas_call(
    _tc_a_body,
    grid=(N // BR,),
    in_specs=[
        _row_spec(D), _row_spec(1), _row_spec(1),
        _full_spec(128, 256), _full_spec(1, 256),
        _full_spec(256, 128), _full_spec(1, 128),
        _full_spec(128, 128),
    ],
    out_specs=[_row_spec(D), _row_spec(1)],
    out_shape=[
        jax.ShapeDtypeStruct((N, D), jnp.float32),
        jax.ShapeDtypeStruct((N, 1), jnp.float32),
    ],
)

_tc_b = pl.pallas_call(
    _tc_b_body,
    grid=(N // BR,),
    in_specs=[
        _row_spec(D), _row_spec(D), _row_spec(D), _row_spec(1),
        _full_spec(1, 128), _full_spec(128, 128),
    ],
    out_specs=_row_spec(D),
    out_shape=jax.ShapeDtypeStruct((N, D), jnp.float32),
)

_tc_c = pl.pallas_call(
    _tc_c_body,
    grid=(N // BR,),
    in_specs=[
        _row_spec(D), _row_spec(D), _row_spec(D), _row_spec(1),
        _full_spec(1, 128),
    ],
    out_specs=_row_spec(D),
    out_shape=jax.ShapeDtypeStruct((N, D), jnp.float32),
)


def kernel(X, edge_index, Wm0, bm0, Wm1, bm1, Wg0, bg0, Wg1, bg1):
    src = edge_index[0]
    dst = edge_index[1]

    degp = _deg_pass(dst)
    d0 = degp[0:N].reshape(N, 1)
    d1 = degp[NPAD:NPAD + N].reshape(N, 1)

    y1, dinv = _tc_a(X, d0, d1, Wm0, bm0.reshape(1, -1), Wm1,
                     bm1.reshape(1, -1), Wg0)

    zp1 = _edge_pass(y1, src, dst)
    y2 = _tc_b(zp1[:N], zp1[NPAD:NPAD + N], y1, dinv, bg0.reshape(1, -1), Wg1)

    zp2 = _edge_pass(y2, src, dst)
    out = _tc_c(zp2[:N], zp2[NPAD:NPAD + N], y2, dinv, bg1.reshape(1, -1))
    return out
